# baseline (speedup 1.0000x reference)
import jax
import jax.numpy as jnp
from jax.experimental import pallas as pl

RB = 256
CHUNK = 2736


def _bf16_round(v):
    return v.astype(jnp.bfloat16).astype(jnp.float32)


def _vq_kernel(x_ref, c_ref, out_ref):
    x = x_ref[...]
    c = c_ref[...]
    k = c.shape[0]
    x2 = jnp.sum(x * x, axis=1, keepdims=True)
    y2 = jnp.sum(c * c, axis=1)[None, :]
    xy = jax.lax.dot_general(
        x, c, (((1,), (1,)), ((), ())),
        preferred_element_type=jnp.float32)
    d2 = (x2 + y2) - 2.0 * xy
    dist = jnp.sqrt(jnp.maximum(d2, 0.0))

    kidx = jax.lax.broadcasted_iota(jnp.int32, dist.shape, 1)

    def chunk_minargmin(lo, hi):
        masked = jnp.where((kidx >= lo) & (kidx < hi), dist, jnp.inf)
        return jnp.min(masked, axis=1), jnp.argmin(masked, axis=1)

    m, idx = chunk_minargmin(0, CHUNK)
    r = _bf16_round(m)
    for lo in range(CHUNK, k, CHUNK):
        m2, i2 = chunk_minargmin(lo, min(lo + CHUNK, k))
        upd = m2 < r
        idx = jnp.where(upd, i2, idx)
        r = _bf16_round(jnp.where(upd, m2, r))
    out_ref[0, 0, :] = idx


def kernel(latents, codebook):
    b, l, d = latents.shape
    k = codebook.shape[0]
    n = b * l
    x = latents.reshape(n, d)
    nblk = n // RB
    out = pl.pallas_call(
        _vq_kernel,
        grid=(nblk,),
        in_specs=[
            pl.BlockSpec((RB, d), lambda i: (i, 0)),
            pl.BlockSpec((k, d), lambda i: (0, 0)),
        ],
        out_specs=pl.BlockSpec((1, 1, RB), lambda i: (i, 0, 0)),
        out_shape=jax.ShapeDtypeStruct((nblk, 1, RB), jnp.int32),
    )(x, codebook)
    return out.reshape(b, l)


# speedup vs baseline: 1.2164x; 1.2164x over previous
import jax
import jax.numpy as jnp
from jax.experimental import pallas as pl

RB = 256
CHUNK = 2736
CPAD = 2816
JUNK = 1e15


def _bf16_round(v):
    return v.astype(jnp.bfloat16).astype(jnp.float32)


def _y2_kernel(c_ref, y2_ref):
    c = c_ref[...]
    y2_ref[...] = jnp.sum(c * c, axis=1)[None, :]


def _vq_kernel(xs_ref, c_ref, y2_ref, out_ref):
    xs = xs_ref[...]
    x2 = 0.25 * jnp.sum(xs * xs, axis=1, keepdims=True)

    def chunk_dot(j):
        cj = c_ref[j * CPAD:(j + 1) * CPAD, :]
        return jax.lax.dot_general(
            xs, cj, (((1,), (1,)), ((), ())),
            preferred_element_type=jnp.float32)

    iota = jax.lax.broadcasted_iota(jnp.int32, (RB, CPAD), 1)
    big = jnp.int32(2 ** 30)

    def chunk_reduce(xy, j):
        y2j = y2_ref[0, j * CPAD:(j + 1) * CPAD][None, :]
        d2 = (x2 + y2j) + xy
        dist = jnp.sqrt(jnp.maximum(d2, 0.0))
        m = jnp.min(dist, axis=1)
        idx = jnp.min(jnp.where(dist == m[:, None], iota, big), axis=1)
        return m, idx

    xy1 = chunk_dot(0)
    xy2 = chunk_dot(1)
    m1, i1 = chunk_reduce(xy1, 0)
    xy3 = chunk_dot(2)
    m2, i2 = chunk_reduce(xy2, 1)
    m3, i3 = chunk_reduce(xy3, 2)

    r = _bf16_round(m1)
    idx = i1
    upd = m2 < r
    idx = jnp.where(upd, i2 + CHUNK, idx)
    r = _bf16_round(jnp.where(upd, m2, r))
    upd = m3 < r
    idx = jnp.where(upd, i3 + 2 * CHUNK, idx)
    out_ref[0, 0, :] = idx


def kernel(latents, codebook):
    b, l, d = latents.shape
    k = codebook.shape[0]
    n = b * l
    xs = latents.reshape(n, d) * -2.0
    junk = jnp.full((CPAD, d), JUNK, dtype=codebook.dtype)
    cpad = jnp.concatenate([
        codebook[0:CHUNK], junk[:CPAD - CHUNK],
        codebook[CHUNK:2 * CHUNK], junk[:CPAD - CHUNK],
        codebook[2 * CHUNK:k], junk[:CPAD - (k - 2 * CHUNK)],
    ])

    y2 = pl.pallas_call(
        _y2_kernel,
        out_shape=jax.ShapeDtypeStruct((1, 3 * CPAD), jnp.float32),
    )(cpad)

    nblk = n // RB
    out = pl.pallas_call(
        _vq_kernel,
        grid=(nblk,),
        in_specs=[
            pl.BlockSpec((RB, d), lambda i: (i, 0)),
            pl.BlockSpec((3 * CPAD, d), lambda i: (0, 0)),
            pl.BlockSpec((1, 3 * CPAD), lambda i: (0, 0)),
        ],
        out_specs=pl.BlockSpec((1, 1, RB), lambda i: (i, 0, 0)),
        out_shape=jax.ShapeDtypeStruct((nblk, 1, RB), jnp.int32),
    )(xs, cpad, y2)
    return out.reshape(b, l)


# speedup vs baseline: 1.5006x; 1.2336x over previous
import jax
import jax.numpy as jnp
from jax.experimental import pallas as pl

RB = 256
XB = 1024
CHUNK = 2736
CPAD = 2816
JUNK = 1e15


def _bf16_round(v):
    return v.astype(jnp.bfloat16).astype(jnp.float32)


def _y2_kernel(c_ref, y2_ref):
    c = c_ref[...]
    y2_ref[...] = jnp.sum(c * c, axis=1)[None, :]


def _x2_kernel(x_ref, x2_ref):
    xs = x_ref[...] * -2.0
    x2_ref[0, 0, :] = 0.25 * jnp.sum(xs * xs, axis=1)


def _vq_kernel(x_ref, c_ref, y2_ref, x2_ref, out_ref):
    xs = x_ref[...] * -2.0
    x2 = x2_ref[0, 0, :][:, None]

    def chunk_dot(j):
        cj = c_ref[j * CPAD:(j + 1) * CPAD, :]
        return jax.lax.dot_general(
            xs, cj, (((1,), (1,)), ((), ())),
            preferred_element_type=jnp.float32)

    iota = jax.lax.broadcasted_iota(jnp.int32, (RB, CPAD), 1)
    big = jnp.int32(2 ** 30)
    inf = jnp.float32(jnp.inf)

    def chunk_reduce(xy, j):
        y2j = y2_ref[0, j * CPAD:(j + 1) * CPAD][None, :]
        d2c = jnp.maximum((x2 + y2j) + xy, 0.0)
        m = jnp.min(d2c, axis=1)
        s = jnp.sqrt(m)
        mb = jax.lax.bitcast_convert_type(m, jnp.int32)
        hi = inf
        for j_ulp in range(1, 6):
            u = jax.lax.bitcast_convert_type(mb + j_ulp, jnp.float32)
            hi = jnp.minimum(hi, jnp.where(jnp.sqrt(u) != s, u, inf))
        idx = jnp.min(jnp.where(d2c < hi[:, None], iota, big), axis=1)
        return s, idx

    xy1 = chunk_dot(0)
    xy2 = chunk_dot(1)
    s1, i1 = chunk_reduce(xy1, 0)
    xy3 = chunk_dot(2)
    s2, i2 = chunk_reduce(xy2, 1)
    s3, i3 = chunk_reduce(xy3, 2)

    r = _bf16_round(s1)
    idx = i1
    upd = s2 < r
    idx = jnp.where(upd, i2 + CHUNK, idx)
    r = _bf16_round(jnp.where(upd, s2, r))
    upd = s3 < r
    idx = jnp.where(upd, i3 + 2 * CHUNK, idx)
    out_ref[0, 0, :] = idx


def kernel(latents, codebook):
    b, l, d = latents.shape
    k = codebook.shape[0]
    n = b * l
    x = latents.reshape(n, d)
    junk = jnp.full((CPAD, d), JUNK, dtype=codebook.dtype)
    cpad = jnp.concatenate([
        codebook[0:CHUNK], junk[:CPAD - CHUNK],
        codebook[CHUNK:2 * CHUNK], junk[:CPAD - CHUNK],
        codebook[2 * CHUNK:k], junk[:CPAD - (k - 2 * CHUNK)],
    ])

    y2 = pl.pallas_call(
        _y2_kernel,
        out_shape=jax.ShapeDtypeStruct((1, 3 * CPAD), jnp.float32),
    )(cpad)

    x2 = pl.pallas_call(
        _x2_kernel,
        grid=(n // XB,),
        in_specs=[pl.BlockSpec((XB, d), lambda i: (i, 0))],
        out_specs=pl.BlockSpec((1, 1, XB), lambda i: (i, 0, 0)),
        out_shape=jax.ShapeDtypeStruct((n // XB, 1, XB), jnp.float32),
    )(x).reshape(n // RB, 1, RB)

    nblk = n // RB
    out = pl.pallas_call(
        _vq_kernel,
        grid=(nblk,),
        in_specs=[
            pl.BlockSpec((RB, d), lambda i: (i, 0)),
            pl.BlockSpec((3 * CPAD, d), lambda i: (0, 0)),
            pl.BlockSpec((1, 3 * CPAD), lambda i: (0, 0)),
            pl.BlockSpec((1, 1, RB), lambda i: (i, 0, 0)),
        ],
        out_specs=pl.BlockSpec((1, 1, RB), lambda i: (i, 0, 0)),
        out_shape=jax.ShapeDtypeStruct((nblk, 1, RB), jnp.int32),
    )(x, cpad, y2, x2)
    return out.reshape(b, l)


# speedup vs baseline: 2.1368x; 1.4240x over previous
import jax
import jax.numpy as jnp
from jax.experimental import pallas as pl
from jax.experimental.pallas import tpu as pltpu

RB = 512
XB = 1024
CHUNK = 2736
K = 8192


def _bf16_round(v):
    return v.astype(jnp.bfloat16).astype(jnp.float32)


def _x2_kernel(x_ref, x2_ref):
    xs = x_ref[...] * -2.0
    x2 = 0.25 * jnp.sum(xs * xs, axis=1, keepdims=True)
    x2_ref[...] = jnp.broadcast_to(x2, (x2.shape[0], 128))


def _vq_kernel(x_ref, c_ref, x2_ref, out_ref, y2_scr):
    i = pl.program_id(0)

    @pl.when(i == 0)
    def _():
        c = c_ref[...]
        y2_scr[...] = jnp.sum(c * c, axis=1)[None, :]

    xs = x_ref[...] * -2.0
    x2 = x2_ref[:, 0:1]
    y2 = y2_scr[...]

    xy = jax.lax.dot_general(
        xs, c_ref[...], (((1,), (1,)), ((), ())),
        preferred_element_type=jnp.float32)

    iota = jax.lax.broadcasted_iota(
        jnp.int32, (1, K), 1).astype(jnp.float32)
    lane = jax.lax.broadcasted_iota(jnp.int32, (1, 128), 1)
    inf = jnp.float32(jnp.inf)

    def cols(a, t):
        return a[:, t * 128:(t + 1) * 128]

    def chunk_min(d2, lo, hi):
        v0, r0 = divmod(lo, 128)
        v1, r1 = divmod(hi, 128)
        acc = None
        for t in range(v0 + (1 if r0 else 0), v1):
            acc = cols(d2, t) if acc is None else jnp.minimum(acc, cols(d2, t))
        if r0:
            acc = jnp.minimum(acc, jnp.where(lane >= r0, cols(d2, v0), inf))
        if r1:
            acc = jnp.minimum(acc, jnp.where(lane < r1, cols(d2, v1), inf))
        return jnp.min(acc, axis=1)

    def chunk_scan(d2, lo, hi, bound):
        b = bound[:, None]
        v0, r0 = divmod(lo, 128)
        v1, r1 = divmod(hi, 128)
        acc = None
        for t in range(v0 + (1 if r0 else 0), v1):
            cand = jnp.where(cols(d2, t) < b, cols(iota, t), inf)
            acc = cand if acc is None else jnp.minimum(acc, cand)
        if r0:
            cand = jnp.where((cols(d2, v0) < b) & (lane >= r0),
                             cols(iota, v0), inf)
            acc = jnp.minimum(acc, cand)
        if r1:
            cand = jnp.where((cols(d2, v1) < b) & (lane < r1),
                             cols(iota, v1), inf)
            acc = jnp.minimum(acc, cand)
        return jnp.min(acc, axis=1)

    d2 = (x2 + y2) + xy

    def chunk_reduce(lo, hi):
        mraw = chunk_min(d2, lo, hi)
        m = jnp.maximum(mraw, 0.0)
        s = jnp.sqrt(m)
        g = s * 4097.0
        sh = g - (g - s)
        sl = s - sh
        p = s * s
        e = ((sh * sh - p) + 2.0 * (sh * sl)) + sl * sl
        sb = jax.lax.bitcast_convert_type(s, jnp.int32)
        h = jax.lax.bitcast_convert_type(
            sb & jnp.int32(0x7F800000), jnp.float32) * jnp.float32(2.0 ** -23)
        t = e + s * h
        mb = jax.lax.bitcast_convert_type(m, jnp.int32)
        npass = jnp.int32(0)
        for j_ulp in range(1, 4):
            u = jax.lax.bitcast_convert_type(mb + j_ulp, jnp.float32)
            npass = npass + ((u - p) <= t).astype(jnp.int32)
        hi_b = jax.lax.bitcast_convert_type(mb + 1 + npass, jnp.float32)
        idx = chunk_scan(d2, lo, hi, hi_b)
        return s, idx

    s1, i1 = chunk_reduce(0, CHUNK)
    s2, i2 = chunk_reduce(CHUNK, 2 * CHUNK)
    s3, i3 = chunk_reduce(2 * CHUNK, K)

    r = _bf16_round(s1)
    idx = i1
    upd = s2 < r
    idx = jnp.where(upd, i2, idx)
    r = _bf16_round(jnp.where(upd, s2, r))
    upd = s3 < r
    idx = jnp.where(upd, i3, idx)
    out_ref[0, 0, :] = idx.astype(jnp.int32)


def kernel(latents, codebook):
    b, l, d = latents.shape
    n = b * l
    x = latents.reshape(n, d)

    x2 = pl.pallas_call(
        _x2_kernel,
        grid=(n // XB,),
        in_specs=[pl.BlockSpec((XB, d), lambda i: (i, 0))],
        out_specs=pl.BlockSpec((XB, 128), lambda i: (i, 0)),
        out_shape=jax.ShapeDtypeStruct((n, 128), jnp.float32),
    )(x)

    nblk = n // RB
    out = pl.pallas_call(
        _vq_kernel,
        grid=(nblk,),
        in_specs=[
            pl.BlockSpec((RB, d), lambda i: (i, 0)),
            pl.BlockSpec((K, d), lambda i: (0, 0)),
            pl.BlockSpec((RB, 128), lambda i: (i, 0)),
        ],
        out_specs=pl.BlockSpec((1, 1, RB), lambda i: (i, 0, 0)),
        out_shape=jax.ShapeDtypeStruct((nblk, 1, RB), jnp.int32),
        scratch_shapes=[pltpu.VMEM((1, K), jnp.float32)],
    )(x, codebook, x2)
    return out.reshape(b, l)
